# baseline (speedup 1.0000x reference)
import functools

import jax
import jax.numpy as jnp
from jax import lax
from jax.experimental import pallas as pl
from jax.experimental.pallas import tpu as pltpu
from jax.experimental.pallas import tpu_sc as plsc

_P, _L, _N, _D = 2000, 32, 50000, 512
_R = _P * _L
_NC, _NS = 2, 16
_SHARD = _R // _NS
_PASSES = 13
_C = 2048
_CT = _C // _NS
_CAP = 4096
_QTR = 800
_PULL = 256
_BIG = 0x40000000
_TAIL_PART = _N % _CT


def _sc_segment_sum(idx, rows):
    mesh = plsc.VectorSubcoreMesh(core_axis_name="c", subcore_axis_name="s")

    @functools.partial(
        pl.kernel,
        out_type=(
            jax.ShapeDtypeStruct((_N, _D), jnp.float32),
            jax.ShapeDtypeStruct((_N, 16), jnp.float32),
        ),
        mesh=mesh,
        compiler_params=pltpu.CompilerParams(needs_layout_passes=False),
        scratch_types=[
            pltpu.VMEM((_QTR,), jnp.int32),
            pltpu.VMEM((_CAP,), jnp.int32),
            pltpu.VMEM((_CAP,), jnp.int32),
            pltpu.VMEM((_PULL,), jnp.int32),
            pltpu.VMEM((_PULL,), jnp.int32),
            pltpu.VMEM((16, 16), jnp.int32),
            pltpu.VMEM((16,), jnp.int32),
            pltpu.VMEM((16, _D), jnp.float32),
            pltpu.VMEM((16, 16), jnp.float32),
            pltpu.VMEM((16, 16), jnp.float32),
            pltpu.VMEM((_CT + 16, _D), jnp.float32),
            pltpu.VMEM((_CT + 16, 16), jnp.float32),
            pltpu.VMEM_SHARED((16, _CAP), jnp.int32),
            pltpu.VMEM_SHARED((16, _CAP), jnp.int32),
            pltpu.VMEM_SHARED((16, 16), jnp.int32),
            pltpu.VMEM_SHARED((16, _D), jnp.float32),
            pltpu.VMEM_SHARED((16, 16), jnp.float32),
            pltpu.SemaphoreType.DMA,
        ],
    )
    def sc_kernel(idx_hbm, rows_hbm, sums_hbm, cnt_hbm,
                  idx_v, cand_pos, cand_off, pull_pos, pull_off,
                  cnt_loc, kbuf, staging, czbuf, ones_v, acc, cnt_acc,
                  mbox_pos, mbox_off, mbox_cnt, zslab, czslab, sem):
        c = lax.axis_index("c")
        s = lax.axis_index("s")
        zero16f = jnp.zeros((16,), jnp.float32)
        one16f = jnp.ones((16,), jnp.float32)
        iota16 = lax.iota(jnp.int32, 16)

        def init_row(i, _):
            czbuf[i, :] = zero16f
            ones_v[i, :] = one16f
            return _
        lax.fori_loop(0, 16, init_row, None)

        def init_z(i, _):
            staging[i // 32, pl.ds((i % 32) * 16, 16)] = zero16f
            return _
        lax.fori_loop(0, 16 * (_D // 16), init_z, None)

        @pl.when(s == 0)
        def _():
            pltpu.sync_copy(staging, zslab)
            pltpu.sync_copy(czbuf, czslab)
        plsc.subcore_barrier()

        def pass_body(p, _):
            base = (c * _PASSES + p) * _C

            def zacc(r, _):
                pltpu.sync_copy(zslab, acc.at[pl.ds(r * 16, 16)])
                return _
            lax.fori_loop(0, (_CT + 16) // 16, zacc, None)

            def zcnt(r, _):
                pltpu.sync_copy(czslab, cnt_acc.at[pl.ds(r * 16, 16)])
                return _
            lax.fori_loop(0, (_CT + 16) // 16, zcnt, None)

            def scan_piece(qt, n):
                pltpu.sync_copy(
                    idx_hbm.at[pl.ds(s * _SHARD + qt * _QTR, _QTR)], idx_v)

                def scan(i, n):
                    idxv = idx_v[pl.ds(i * 16, 16)]
                    off = idxv - base
                    inr = (off >= 0) & (off < _C)
                    posv = s * _SHARD + qt * _QTR + i * 16 + iota16
                    cnt_in = inr.astype(jnp.int32)
                    pref = plsc.cumsum(cnt_in)
                    dest = jnp.where(inr, n + pref - 1, _CAP - 16 + iota16)
                    plsc.store_scatter(cand_pos, [dest], posv)
                    plsc.store_scatter(cand_off, [dest], off)
                    return n + jnp.sum(cnt_in)
                return lax.fori_loop(0, _QTR // 16, scan, n)
            k = lax.fori_loop(0, _SHARD // _QTR, scan_piece, jnp.int32(0))

            pltpu.sync_copy(cand_pos, mbox_pos.at[s])
            pltpu.sync_copy(cand_off, mbox_off.at[s])
            kbuf[...] = jnp.broadcast_to(k, (16,))
            pltpu.sync_copy(kbuf, mbox_cnt.at[s])
            plsc.subcore_barrier()

            pltpu.sync_copy(mbox_cnt, cnt_loc)
            my_lo = s * _CT

            def per_scanner(sc, _):
                kc = jnp.max(cnt_loc[sc, :])

                def pull(ch, m):
                    pltpu.sync_copy(mbox_pos.at[sc, pl.ds(ch * _PULL, _PULL)],
                                    pull_pos)
                    pltpu.sync_copy(mbox_off.at[sc, pl.ds(ch * _PULL, _PULL)],
                                    pull_off)

                    def filt(q, m):
                        g = ch * _PULL + q * 16
                        pv = pull_pos[pl.ds(q * 16, 16)]
                        ov = pull_off[pl.ds(q * 16, 16)]
                        lo = ov - my_lo
                        mine = ((g + iota16 < kc) & (lo >= 0) & (lo < _CT))
                        cnt_in = mine.astype(jnp.int32)
                        pref = plsc.cumsum(cnt_in)
                        dest = jnp.where(mine, m + pref - 1,
                                         _CAP - 16 + iota16)
                        plsc.store_scatter(cand_pos, [dest], pv)
                        plsc.store_scatter(cand_off, [dest], lo)
                        return m + jnp.sum(cnt_in)
                    return lax.fori_loop(0, _PULL // 16, filt, m)
                m = lax.fori_loop(0, (kc + _PULL - 1) // _PULL, pull,
                                  jnp.int32(0))

                cand_pos[pl.ds(m, 16)] = jnp.zeros((16,), jnp.int32)
                cand_off[pl.ds(m, 16)] = jnp.full((16,), _CT, jnp.int32)

                def chunk(g, _):
                    fpos = cand_pos[pl.ds(g * 16, 16)]
                    pltpu.async_copy(rows_hbm.at[fpos], staging, sem).wait()

                    def row(q, _):
                        ov = cand_off[pl.ds(g * 16, 16)]
                        o = jnp.sum(jnp.where(iota16 == q, ov, 0))
                        for cc in range(_D // 16):
                            sl = pl.ds(cc * 16, 16)
                            plsc.addupdate(acc.at[o, sl], staging[q, sl])
                        plsc.addupdate(cnt_acc.at[o], ones_v[0, :])
                        return _
                    lax.fori_loop(0, 16, row, None)
                    return _
                lax.fori_loop(0, (m + 15) // 16, chunk, None)
                return _
            lax.fori_loop(0, _NS, per_scanner, None)

            gbase = base + s * _CT

            @pl.when(gbase + _CT <= _N)
            def _full():
                pltpu.sync_copy(acc.at[pl.ds(0, _CT)],
                                sums_hbm.at[pl.ds(gbase, _CT)])
                pltpu.sync_copy(cnt_acc.at[pl.ds(0, _CT)],
                                cnt_hbm.at[pl.ds(gbase, _CT)])

            if _TAIL_PART > 0:
                @pl.when(jnp.logical_and(gbase < _N, gbase + _CT > _N))
                def _part():
                    pltpu.sync_copy(acc.at[pl.ds(0, _TAIL_PART)],
                                    sums_hbm.at[pl.ds(gbase, _TAIL_PART)])
                    pltpu.sync_copy(cnt_acc.at[pl.ds(0, _TAIL_PART)],
                                    cnt_hbm.at[pl.ds(gbase, _TAIL_PART)])

            plsc.subcore_barrier()
            return _

        lax.fori_loop(0, _PASSES, pass_body, None)

    return sc_kernel(idx, rows)


_BM = 400


def _tc_body(prev_ref, sums_ref, cnt_ref, nmask_ref, w1_ref, w2_ref, b_ref,
             out_ref):
    cnt = cnt_ref[:, 0:1]
    mean = sums_ref[...] / jnp.maximum(cnt, 1.0)
    dn = (((1,), (1,)), ((), ()))
    y = lax.dot_general(prev_ref[...], w1_ref[...], dn,
                        preferred_element_type=jnp.float32)
    y = y + lax.dot_general(mean, w2_ref[...], dn,
                            preferred_element_type=jnp.float32)
    y = jnp.maximum(y + b_ref[...], 0.0)
    out_ref[...] = jnp.where(nmask_ref[...] != 0.0, y, prev_ref[...])


def _tc_combine(prev, sums, cnt2d, nmask, w1, w2, b2d):
    n, d = prev.shape
    grid = (n // _BM,)
    return pl.pallas_call(
        _tc_body,
        grid=grid,
        in_specs=[
            pl.BlockSpec((_BM, d), lambda i: (i, 0)),
            pl.BlockSpec((_BM, d), lambda i: (i, 0)),
            pl.BlockSpec((_BM, 16), lambda i: (i, 0)),
            pl.BlockSpec((_BM, 1), lambda i: (i, 0)),
            pl.BlockSpec((d, d), lambda i: (0, 0)),
            pl.BlockSpec((d, d), lambda i: (0, 0)),
            pl.BlockSpec((1, d), lambda i: (0, 0)),
        ],
        out_specs=pl.BlockSpec((_BM, d), lambda i: (i, 0)),
        out_shape=jax.ShapeDtypeStruct((n, d), jnp.float32),
    )(prev, sums, cnt2d, nmask, w1, w2, b2d)


def kernel(encoded_cfg_paths, cfg_paths_mask, cfg_paths_node_indices,
           previous_cfg_nodes_encodings, cfg_nodes_has_expression_mask, W, b):
    n, d = previous_cfg_nodes_encodings.shape
    rows = encoded_cfg_paths.reshape(_R, d)
    idx = jnp.where(cfg_paths_mask.reshape(_R),
                    cfg_paths_node_indices.reshape(_R), jnp.int32(_BIG))

    sums, cnt2d = _sc_segment_sum(idx, rows)

    w1 = W[:, :d]
    w2 = W[:, d:]
    b2d = b.reshape(1, d)
    nmask = cfg_nodes_has_expression_mask.astype(jnp.float32).reshape(n, 1)
    return _tc_combine(previous_cfg_nodes_encodings, sums, cnt2d, nmask,
                       w1, w2, b2d)


# speedup vs baseline: 2.0085x; 2.0085x over previous
import functools

import jax
import jax.numpy as jnp
from jax import lax
from jax.experimental import pallas as pl
from jax.experimental.pallas import tpu as pltpu
from jax.experimental.pallas import tpu_sc as plsc

_P, _L, _N, _D = 2000, 32, 50000, 512
_R = _P * _L
_NC, _NS = 2, 16
_SHARD = _R // _NS
_PASSES = 13
_C = 2048
_CT = _C // _NS
_CAP = 4096
_QTR = 800
_PULL = 256
_BIG = 0x40000000
_TAIL_PART = _N % _CT


def _sc_segment_sum(idx, rows):
    mesh = plsc.VectorSubcoreMesh(core_axis_name="c", subcore_axis_name="s")

    @functools.partial(
        pl.kernel,
        out_type=(
            jax.ShapeDtypeStruct((_N, _D), jnp.float32),
            jax.ShapeDtypeStruct((_N, 16), jnp.float32),
        ),
        mesh=mesh,
        compiler_params=pltpu.CompilerParams(needs_layout_passes=False),
        scratch_types=[
            pltpu.VMEM((_QTR,), jnp.int32),
            pltpu.VMEM((_CAP,), jnp.int32),
            pltpu.VMEM((_CAP,), jnp.int32),
            pltpu.VMEM((_PULL,), jnp.int32),
            pltpu.VMEM((_PULL,), jnp.int32),
            pltpu.VMEM((16, 16), jnp.int32),
            pltpu.VMEM((16,), jnp.int32),
            pltpu.VMEM((16, _D), jnp.float32),
            pltpu.VMEM((16, 16), jnp.float32),
            pltpu.VMEM((16, 16), jnp.float32),
            pltpu.VMEM((_CT + 16, _D), jnp.float32),
            pltpu.VMEM((_CT + 16, 16), jnp.float32),
            pltpu.VMEM_SHARED((16, _CAP), jnp.int32),
            pltpu.VMEM_SHARED((16, _CAP), jnp.int32),
            pltpu.VMEM_SHARED((16, 16), jnp.int32),
            pltpu.VMEM_SHARED((16, _D), jnp.float32),
            pltpu.VMEM_SHARED((16, 16), jnp.float32),
            pltpu.SemaphoreType.DMA,
        ],
    )
    def sc_kernel(idx_hbm, rows_hbm, sums_hbm, cnt_hbm,
                  idx_v, cand_pos, cand_off, pull_pos, pull_off,
                  cnt_loc, kbuf, staging, czbuf, ones_v, acc, cnt_acc,
                  mbox_pos, mbox_off, mbox_cnt, zslab, czslab, sem):
        c = lax.axis_index("c")
        s = lax.axis_index("s")
        zero16f = jnp.zeros((16,), jnp.float32)
        one16f = jnp.ones((16,), jnp.float32)
        iota16 = lax.iota(jnp.int32, 16)

        def init_row(i, _):
            czbuf[i, :] = zero16f
            ones_v[i, :] = one16f
            return _
        lax.fori_loop(0, 16, init_row, None)

        def init_z(i, _):
            staging[i // 32, pl.ds((i % 32) * 16, 16)] = zero16f
            return _
        lax.fori_loop(0, 16 * (_D // 16), init_z, None)

        @pl.when(s == 0)
        def _():
            pltpu.sync_copy(staging, zslab)
            pltpu.sync_copy(czbuf, czslab)
        plsc.subcore_barrier()

        def pass_body(p, _):
            base = (c * _PASSES + p) * _C

            def zacc(r, _):
                pltpu.sync_copy(zslab, acc.at[pl.ds(r * 16, 16)])
                return _
            lax.fori_loop(0, (_CT + 16) // 16, zacc, None)

            def zcnt(r, _):
                pltpu.sync_copy(czslab, cnt_acc.at[pl.ds(r * 16, 16)])
                return _
            lax.fori_loop(0, (_CT + 16) // 16, zcnt, None)

            def scan_piece(qt, n):
                pltpu.sync_copy(
                    idx_hbm.at[pl.ds(s * _SHARD + qt * _QTR, _QTR)], idx_v)

                def scan(i, n):
                    idxv = idx_v[pl.ds(i * 16, 16)]
                    off = idxv - base
                    inr = (off >= 0) & (off < _C)
                    posv = s * _SHARD + qt * _QTR + i * 16 + iota16
                    cnt_in = inr.astype(jnp.int32)
                    pref = plsc.cumsum(cnt_in)
                    dest = jnp.where(inr, n + pref - 1, _CAP - 16 + iota16)
                    plsc.store_scatter(cand_pos, [dest], posv)
                    plsc.store_scatter(cand_off, [dest], off)
                    return n + pref[15]
                return lax.fori_loop(0, _QTR // 16, scan, n)
            k = lax.fori_loop(0, _SHARD // _QTR, scan_piece, jnp.int32(0))

            pltpu.sync_copy(cand_pos, mbox_pos.at[s])
            pltpu.sync_copy(cand_off, mbox_off.at[s])
            kbuf[...] = jnp.broadcast_to(k, (16,))
            pltpu.sync_copy(kbuf, mbox_cnt.at[s])
            plsc.subcore_barrier()

            pltpu.sync_copy(mbox_cnt, cnt_loc)
            my_lo = s * _CT

            def drain(m):
                cand_pos[pl.ds(m, 16)] = jnp.zeros((16,), jnp.int32)
                cand_off[pl.ds(m, 16)] = jnp.full((16,), _CT, jnp.int32)

                def chunk(g, _):
                    fpos = cand_pos[pl.ds(g * 16, 16)]
                    pltpu.async_copy(rows_hbm.at[fpos], staging, sem).wait()
                    ov = cand_off[pl.ds(g * 16, 16)]

                    def row(q, _):
                        o = jnp.sum(jnp.where(iota16 == q, ov, 0))
                        for cc in range(_D // 16):
                            sl = pl.ds(cc * 16, 16)
                            plsc.addupdate(acc.at[o, sl], staging[q, sl])
                        plsc.addupdate(cnt_acc.at[o], ones_v[0, :])
                        return _
                    lax.fori_loop(0, 16, row, None)
                    return _
                lax.fori_loop(0, (m + 15) // 16, chunk, None)

            def per_scanner(sc, m):
                kc = jnp.max(cnt_loc[sc, :])
                need_flush = jnp.logical_and(m > 0, m + kc > _CAP - 16)

                @pl.when(need_flush)
                def _():
                    drain(m)
                m = jnp.where(need_flush, jnp.int32(0), m)

                def pull(ch, m):
                    pltpu.sync_copy(mbox_pos.at[sc, pl.ds(ch * _PULL, _PULL)],
                                    pull_pos)
                    pltpu.sync_copy(mbox_off.at[sc, pl.ds(ch * _PULL, _PULL)],
                                    pull_off)

                    def filt(q, m):
                        g = ch * _PULL + q * 16
                        pv = pull_pos[pl.ds(q * 16, 16)]
                        ov = pull_off[pl.ds(q * 16, 16)]
                        lo = ov - my_lo
                        mine = ((g + iota16 < kc) & (lo >= 0) & (lo < _CT))
                        cnt_in = mine.astype(jnp.int32)
                        pref = plsc.cumsum(cnt_in)
                        dest = jnp.where(mine, m + pref - 1,
                                         _CAP - 16 + iota16)
                        plsc.store_scatter(cand_pos, [dest], pv)
                        plsc.store_scatter(cand_off, [dest], lo)
                        return m + pref[15]
                    return lax.fori_loop(0, _PULL // 16, filt, m)
                return lax.fori_loop(0, (kc + _PULL - 1) // _PULL, pull, m)
            m = lax.fori_loop(0, _NS, per_scanner, jnp.int32(0))

            @pl.when(m > 0)
            def _():
                drain(m)

            gbase = base + s * _CT

            @pl.when(gbase + _CT <= _N)
            def _full():
                pltpu.sync_copy(acc.at[pl.ds(0, _CT)],
                                sums_hbm.at[pl.ds(gbase, _CT)])
                pltpu.sync_copy(cnt_acc.at[pl.ds(0, _CT)],
                                cnt_hbm.at[pl.ds(gbase, _CT)])

            if _TAIL_PART > 0:
                @pl.when(jnp.logical_and(gbase < _N, gbase + _CT > _N))
                def _part():
                    pltpu.sync_copy(acc.at[pl.ds(0, _TAIL_PART)],
                                    sums_hbm.at[pl.ds(gbase, _TAIL_PART)])
                    pltpu.sync_copy(cnt_acc.at[pl.ds(0, _TAIL_PART)],
                                    cnt_hbm.at[pl.ds(gbase, _TAIL_PART)])

            plsc.subcore_barrier()
            return _

        lax.fori_loop(0, _PASSES, pass_body, None)

    return sc_kernel(idx, rows)


_BM = 400


def _tc_body(prev_ref, sums_ref, cnt_ref, nmask_ref, w1_ref, w2_ref, b_ref,
             out_ref):
    cnt = cnt_ref[:, 0:1]
    mean = sums_ref[...] / jnp.maximum(cnt, 1.0)
    dn = (((1,), (1,)), ((), ()))
    y = lax.dot_general(prev_ref[...], w1_ref[...], dn,
                        preferred_element_type=jnp.float32)
    y = y + lax.dot_general(mean, w2_ref[...], dn,
                            preferred_element_type=jnp.float32)
    y = jnp.maximum(y + b_ref[...], 0.0)
    out_ref[...] = jnp.where(nmask_ref[...] != 0.0, y, prev_ref[...])


def _tc_combine(prev, sums, cnt2d, nmask, w1, w2, b2d):
    n, d = prev.shape
    grid = (n // _BM,)
    return pl.pallas_call(
        _tc_body,
        grid=grid,
        in_specs=[
            pl.BlockSpec((_BM, d), lambda i: (i, 0)),
            pl.BlockSpec((_BM, d), lambda i: (i, 0)),
            pl.BlockSpec((_BM, 16), lambda i: (i, 0)),
            pl.BlockSpec((_BM, 1), lambda i: (i, 0)),
            pl.BlockSpec((d, d), lambda i: (0, 0)),
            pl.BlockSpec((d, d), lambda i: (0, 0)),
            pl.BlockSpec((1, d), lambda i: (0, 0)),
        ],
        out_specs=pl.BlockSpec((_BM, d), lambda i: (i, 0)),
        out_shape=jax.ShapeDtypeStruct((n, d), jnp.float32),
    )(prev, sums, cnt2d, nmask, w1, w2, b2d)


def kernel(encoded_cfg_paths, cfg_paths_mask, cfg_paths_node_indices,
           previous_cfg_nodes_encodings, cfg_nodes_has_expression_mask, W, b):
    n, d = previous_cfg_nodes_encodings.shape
    rows = encoded_cfg_paths.reshape(_R, d)
    idx = jnp.where(cfg_paths_mask.reshape(_R),
                    cfg_paths_node_indices.reshape(_R), jnp.int32(_BIG))

    sums, cnt2d = _sc_segment_sum(idx, rows)

    w1 = W[:, :d]
    w2 = W[:, d:]
    b2d = b.reshape(1, d)
    nmask = cfg_nodes_has_expression_mask.astype(jnp.float32).reshape(n, 1)
    return _tc_combine(previous_cfg_nodes_encodings, sums, cnt2d, nmask,
                       w1, w2, b2d)
